# Initial kernel scaffold; baseline (speedup 1.0000x reference)
import jax
import jax.numpy as jnp
from jax.experimental import pallas as pl


def kernel(y, x, local_emb_table, time_w, time_b, vt_W, vt_b, space_table, given_table):
    raise NotImplementedError("write your pallas kernel here")


# speedup vs baseline: 8.1037x; 8.1037x over previous
import jax
import jax.numpy as jnp
from jax.experimental import pallas as pl
from jax.experimental.pallas import tpu as pltpu


def _tc_body(y_ref, x_ref, loc_ref, W2_ref, bf_ref, W1_ref, w0_ref, c_ref,
             d_ref, sp_ref, ovt_ref, osp_ref):
    xb = x_ref[0]
    xb = jnp.where(jnp.isnan(xb), 0.0, xb)
    xa = jnp.dot(xb, W2_ref[...], preferred_element_type=jnp.float32)
    xa = xa + bf_ref[...]
    k = jax.lax.broadcasted_iota(jnp.int32, xa.shape, 1) % 6
    feat = jnp.where(k == 0, xa, jnp.sin(xa))
    tp = jnp.dot(feat, W1_ref[...], preferred_element_type=jnp.float32)
    base = loc_ref[...] + tp + c_ref[...]
    yb = y_ref[0]
    nanm = jnp.isnan(yb)
    y0 = jnp.where(nanm, 0.0, yb)
    nf = nanm.astype(jnp.float32)
    w0 = w0_ref[...]
    dl = d_ref[...]
    for i in range(8):
        ovt_ref[0, i] = base + y0[:, i:i + 1] * w0 + nf[:, i:i + 1] * dl
        osp_ref[0, i] = jnp.broadcast_to(sp_ref[i:i + 1, :], base.shape)


def kernel(y, x, local_emb_table, time_w, time_b, vt_W, vt_b, space_table,
           given_table):
    bs, L, d_y = y.shape
    d_x = x.shape[-1]
    d_model = local_emb_table.shape[-1]
    E = time_w.shape[1]
    TD = d_x * E

    W2 = (jnp.eye(d_x, dtype=jnp.float32)[:, :, None]
          * time_w[None, :, :]).reshape(d_x, TD)
    b_flat = time_b.reshape(1, TD)
    vt_W1 = vt_W[1:]
    w0row = vt_W[0:1]
    crow = (vt_b + given_table[1])[None, :]
    drow = (given_table[0] - given_table[1])[None, :]

    TB = 256
    nt = L // TB
    grid = (bs, nt)
    out4 = [jax.ShapeDtypeStruct((bs, d_y, L, d_model), jnp.float32)] * 2

    vt4, sp4 = pl.pallas_call(
        _tc_body,
        grid=grid,
        in_specs=[
            pl.BlockSpec((1, TB, d_y), lambda b, t: (b, t, 0)),
            pl.BlockSpec((1, TB, d_x), lambda b, t: (b, t, 0)),
            pl.BlockSpec((TB, d_model), lambda b, t: (t, 0)),
            pl.BlockSpec((d_x, TD), lambda b, t: (0, 0)),
            pl.BlockSpec((1, TD), lambda b, t: (0, 0)),
            pl.BlockSpec((TD, d_model), lambda b, t: (0, 0)),
            pl.BlockSpec((1, d_model), lambda b, t: (0, 0)),
            pl.BlockSpec((1, d_model), lambda b, t: (0, 0)),
            pl.BlockSpec((1, d_model), lambda b, t: (0, 0)),
            pl.BlockSpec((d_y, d_model), lambda b, t: (0, 0)),
        ],
        out_specs=[
            pl.BlockSpec((1, d_y, TB, d_model), lambda b, t: (b, 0, t, 0)),
            pl.BlockSpec((1, d_y, TB, d_model), lambda b, t: (b, 0, t, 0)),
        ],
        out_shape=out4,
        compiler_params=pltpu.CompilerParams(
            dimension_semantics=("parallel", "parallel")),
    )(y, x, local_emb_table, W2, b_flat, vt_W1, w0row, crow, drow,
      space_table)

    return (vt4.reshape(bs, d_y * L, d_model),
            sp4.reshape(bs, d_y * L, d_model))
